# Initial kernel scaffold; baseline (speedup 1.0000x reference)
import jax
import jax.numpy as jnp
from jax import lax
from jax.experimental import pallas as pl
from jax.experimental.pallas import tpu as pltpu
from jax.experimental.pallas import tpu_sc as plsc

VOCAB = 1000000
DIM = 32
B = 4096
L = 200

NC = 2
NS = 16
NW = NC * NS

BTOT = B * L
B_PER_W = BTOT // NW
CHUNK = 1280
NCHUNKS = B_PER_W // CHUNK
NBUF = 2
NSTEPS = NCHUNKS // NBUF


def _gather_body(table_hbm, idx_hbm, out_hbm, idx_v, rows_v, gsem):
    wid = lax.axis_index("s") * NC + lax.axis_index("c")
    base = wid * B_PER_W

    pltpu.sync_copy(idx_hbm.at[wid], idx_v)

    def start_gather(c, b):
        pltpu.make_async_copy(
            table_hbm.at[idx_v.at[c]], rows_v.at[b], gsem.at[b]
        ).start()

    def drain_chunk(c, b):
        pltpu.make_async_copy(
            table_hbm.at[idx_v.at[c]], rows_v.at[b], gsem.at[b]
        ).wait()
        pltpu.sync_copy(rows_v.at[b], out_hbm.at[pl.ds(base + c * CHUNK, CHUNK)])

    for b in range(NBUF):
        start_gather(b, b)

    @pl.loop(0, NSTEPS - 1)
    def _steady(i):
        for b in range(NBUF):
            c = i * NBUF + b
            drain_chunk(c, b)
            start_gather(c + NBUF, b)

    for b in range(NBUF):
        drain_chunk((NSTEPS - 1) * NBUF + b, b)


@jax.jit
def _sc_gather(table, idx3d):
    kfn = pl.kernel(
        _gather_body,
        out_type=jax.ShapeDtypeStruct((BTOT, DIM), jnp.float32),
        mesh=plsc.VectorSubcoreMesh(core_axis_name="c", subcore_axis_name="s"),
        scratch_types=[
            pltpu.VMEM((NCHUNKS, CHUNK), jnp.int32),
            pltpu.VMEM((NBUF, CHUNK, DIM), jnp.float32),
            pltpu.SemaphoreType.DMA((NBUF,)),
        ],
    )
    return kfn(table, idx3d)


def kernel(inputs, table):
    idx3d = inputs.astype(jnp.int32).reshape(NW, NCHUNKS, CHUNK)
    out = _sc_gather(table, idx3d)
    return out.reshape(B, L, DIM)


# speedup vs baseline: 1.5021x; 1.5021x over previous
import jax
import jax.numpy as jnp
from jax import lax
from jax.experimental import pallas as pl
from jax.experimental.pallas import tpu as pltpu
from jax.experimental.pallas import tpu_sc as plsc

VOCAB = 1000000
DIM = 32
B = 4096
L = 200

NC = 2
NS = 16
NW = NC * NS

BTOT = B * L
B_PER_W = BTOT // NW
CHUNK = 1280
NCHUNKS = B_PER_W // CHUNK
NBUF = 2
NSTEPS = NCHUNKS // NBUF


def _gather_body(table_hbm, idx_hbm, out_hbm, idx_v, rows_v, gsem):
    wid = lax.axis_index("s") * NC + lax.axis_index("c")
    base = wid * B_PER_W

    pltpu.sync_copy(idx_hbm.at[wid], idx_v)

    def start_gather(c, b):
        pltpu.make_async_copy(
            table_hbm.at[idx_v.at[c]], rows_v.at[b], gsem.at[b]
        ).start()

    def drain_chunk(c, b):
        pltpu.make_async_copy(
            table_hbm.at[idx_v.at[c]], rows_v.at[b], gsem.at[b]
        ).wait()
        pltpu.sync_copy(rows_v.at[b], out_hbm.at[pl.ds(base + c * CHUNK, CHUNK)])

    for b in range(NBUF):
        start_gather(b, b)

    @pl.loop(0, NSTEPS - 1)
    def _steady(i):
        for b in range(NBUF):
            c = i * NBUF + b
            drain_chunk(c, b)
            start_gather(c + NBUF, b)

    for b in range(NBUF):
        drain_chunk((NSTEPS - 1) * NBUF + b, b)


@jax.jit
def _sc_gather(table, idx3d):
    kfn = pl.kernel(
        _gather_body,
        out_type=jax.ShapeDtypeStruct((BTOT, DIM), jnp.float32),
        mesh=plsc.VectorSubcoreMesh(core_axis_name="c", subcore_axis_name="s"),
        scratch_types=[
            pltpu.VMEM((NCHUNKS, CHUNK), jnp.int32),
            pltpu.VMEM((NBUF, CHUNK, DIM), jnp.float32),
            pltpu.SemaphoreType.DMA((NBUF,)),
        ],
        compiler_params=pltpu.CompilerParams(use_tc_tiling_on_sc=False),
    )
    return kfn(table, idx3d)


def kernel(inputs, table):
    idx3d = inputs.astype(jnp.int32).reshape(NW, NCHUNKS, CHUNK)
    out = _sc_gather(table, idx3d)
    return out.reshape(B, L, DIM)
